# baseline (speedup 1.0000x reference)
import numpy as np
import jax
import jax.numpy as jnp
from jax.experimental import pallas as pl
from jax.experimental.pallas import tpu as pltpu

_TIMESTEPS = 1000
_BETAS = np.linspace(0.0001, 0.02, _TIMESTEPS, dtype=np.float64)
_AC = np.cumprod(1.0 - _BETAS)
_TABLES = np.stack([np.sqrt(_AC), np.sqrt(1.0 - _AC)]).astype(np.float32)

_LANES = 256
_ROWS_PER_BATCH = 3 * 256
_BATCHES_PER_BLOCK = 4
_BLOCK_ROWS = _ROWS_PER_BATCH * _BATCHES_PER_BLOCK


def _fma_body(t_hbm, tab_hbm, x_ref, n_ref, o_ref, t_s, tab_s, sem_t, sem_tab):
    blk = pl.program_id(0)

    @pl.when(blk == 0)
    def _():
        t_dma = pltpu.make_async_copy(t_hbm, t_s, sem_t)
        tab_dma = pltpu.make_async_copy(tab_hbm, tab_s, sem_tab)
        t_dma.start()
        tab_dma.start()
        t_dma.wait()
        tab_dma.wait()

    for j in range(_BATCHES_PER_BLOCK):
        bidx = blk * _BATCHES_PER_BLOCK + j
        tt = t_s[bidx]
        a = tab_s[0, tt]
        b = tab_s[1, tt]
        sl = pl.ds(j * _ROWS_PER_BATCH, _ROWS_PER_BATCH)
        o_ref[sl, :] = a * x_ref[sl, :] + b * n_ref[sl, :]


def kernel(x_start, t, noise):
    batch = x_start.shape[0]
    rows = batch * _ROWS_PER_BATCH
    x2 = jnp.transpose(x_start, (0, 3, 1, 2)).reshape(rows, _LANES)
    n2 = jnp.transpose(noise, (0, 3, 1, 2)).reshape(rows, _LANES)
    grid = (rows // _BLOCK_ROWS,)
    out = pl.pallas_call(
        _fma_body,
        grid=grid,
        in_specs=[
            pl.BlockSpec(memory_space=pltpu.MemorySpace.HBM),
            pl.BlockSpec(memory_space=pltpu.MemorySpace.HBM),
            pl.BlockSpec((_BLOCK_ROWS, _LANES), lambda i: (i, 0)),
            pl.BlockSpec((_BLOCK_ROWS, _LANES), lambda i: (i, 0)),
        ],
        out_specs=pl.BlockSpec((_BLOCK_ROWS, _LANES), lambda i: (i, 0)),
        out_shape=jax.ShapeDtypeStruct((rows, _LANES), jnp.float32),
        scratch_shapes=[
            pltpu.SMEM((batch,), jnp.int32),
            pltpu.SMEM((2, _TIMESTEPS), jnp.float32),
            pltpu.SemaphoreType.DMA,
            pltpu.SemaphoreType.DMA,
        ],
    )(t.astype(jnp.int32), jnp.asarray(_TABLES), x2, n2)
    out = out.reshape(batch, 3, 256, 256)
    return jnp.transpose(out, (0, 2, 3, 1))


# speedup vs baseline: 1.0262x; 1.0262x over previous
import numpy as np
import jax
import jax.numpy as jnp
from jax.experimental import pallas as pl
from jax.experimental.pallas import tpu as pltpu

_TIMESTEPS = 1000
_BETAS = np.linspace(0.0001, 0.02, _TIMESTEPS, dtype=np.float64)
_AC = np.cumprod(1.0 - _BETAS)
_TABLES = np.stack([np.sqrt(_AC), np.sqrt(1.0 - _AC)]).astype(np.float32)

_LANES = 256
_ROWS_PER_BATCH = 3 * 256
_BATCHES_PER_BLOCK = 4
_BLOCK_ROWS = _ROWS_PER_BATCH * _BATCHES_PER_BLOCK


def _fma_body(t_ref, tab_ref, x_ref, n_ref, o_ref):
    blk = pl.program_id(0)
    for j in range(_BATCHES_PER_BLOCK):
        bidx = blk * _BATCHES_PER_BLOCK + j
        tt = t_ref[bidx]
        a = tab_ref[0, tt]
        b = tab_ref[1, tt]
        sl = pl.ds(j * _ROWS_PER_BATCH, _ROWS_PER_BATCH)
        o_ref[sl, :] = a * x_ref[sl, :] + b * n_ref[sl, :]


def kernel(x_start, t, noise):
    batch = x_start.shape[0]
    rows = batch * _ROWS_PER_BATCH
    x2 = jnp.transpose(x_start, (0, 3, 1, 2)).reshape(rows, _LANES)
    n2 = jnp.transpose(noise, (0, 3, 1, 2)).reshape(rows, _LANES)
    grid = (rows // _BLOCK_ROWS,)
    out = pl.pallas_call(
        _fma_body,
        grid=grid,
        in_specs=[
            pl.BlockSpec(memory_space=pltpu.SMEM),
            pl.BlockSpec(memory_space=pltpu.SMEM),
            pl.BlockSpec((_BLOCK_ROWS, _LANES), lambda i: (i, 0)),
            pl.BlockSpec((_BLOCK_ROWS, _LANES), lambda i: (i, 0)),
        ],
        out_specs=pl.BlockSpec((_BLOCK_ROWS, _LANES), lambda i: (i, 0)),
        out_shape=jax.ShapeDtypeStruct((rows, _LANES), jnp.float32),
    )(t.astype(jnp.int32), jnp.asarray(_TABLES), x2, n2)
    out = out.reshape(batch, 3, 256, 256)
    return jnp.transpose(out, (0, 2, 3, 1))
